# baseline (speedup 1.0000x reference)
import functools

import jax
import jax.numpy as jnp
from jax import lax
from jax.experimental import pallas as pl
from jax.experimental.pallas import tpu as pltpu
from jax.experimental.pallas import tpu_sc as plsc

B = 4096
SEQ = 200
DIM = 64
ROW = SEQ * DIM
L = 16
NC = 2
NS = 16
NW = NC * NS
PER_W = B // NW
NBUF = 4
GROUPS = PER_W // NBUF
UNROLL = 8
CHUNKS = ROW // (L * UNROLL)

_mesh = plsc.VectorSubcoreMesh(core_axis_name="c", subcore_axis_name="s")


@functools.partial(
    pl.kernel,
    out_type=jax.ShapeDtypeStruct((B, ROW), jnp.float32),
    mesh=_mesh,
    scratch_types=dict(
        pos_v=pltpu.VMEM((ROW,), jnp.float32),
        b0=pltpu.VMEM((ROW,), jnp.float32),
        b1=pltpu.VMEM((ROW,), jnp.float32),
        b2=pltpu.VMEM((ROW,), jnp.float32),
        b3=pltpu.VMEM((ROW,), jnp.float32),
        isem0=pltpu.SemaphoreType.DMA,
        isem1=pltpu.SemaphoreType.DMA,
        isem2=pltpu.SemaphoreType.DMA,
        isem3=pltpu.SemaphoreType.DMA,
        osem0=pltpu.SemaphoreType.DMA,
        osem1=pltpu.SemaphoreType.DMA,
        osem2=pltpu.SemaphoreType.DMA,
        osem3=pltpu.SemaphoreType.DMA,
    ),
)
def _sc_add(x_hbm, pos_hbm, out_hbm, *, pos_v, b0, b1, b2, b3,
            isem0, isem1, isem2, isem3, osem0, osem1, osem2, osem3):
    bufs = [b0, b1, b2, b3]
    isems = [isem0, isem1, isem2, isem3]
    osems = [osem0, osem1, osem2, osem3]

    wid = lax.axis_index("s") * NC + lax.axis_index("c")
    base = wid * PER_W

    pltpu.sync_copy(pos_hbm, pos_v)

    def compute(buf):
        def body(i, carry):
            for j in range(UNROLL):
                sl = pl.ds(i * (L * UNROLL) + j * L, L)
                plsc.addupdate(buf.at[sl], pos_v[sl])
            return carry
        lax.fori_loop(0, CHUNKS, body, 0)

    def slot(g, k, *, osem_wait=True, prefetch=True):
        t = base + g * NBUF + k
        pltpu.make_async_copy(x_hbm.at[t], bufs[k], isems[k]).wait()
        compute(bufs[k])
        pltpu.async_copy(bufs[k], out_hbm.at[t], osems[k])
        if prefetch:
            k2 = (k + 2) % NBUF
            if osem_wait:
                pltpu.make_async_copy(bufs[k2], out_hbm.at[t - 2], osems[k2]).wait()
            pltpu.async_copy(x_hbm.at[t + 2], bufs[k2], isems[k2])

    pltpu.async_copy(x_hbm.at[base + 0], bufs[0], isems[0])
    pltpu.async_copy(x_hbm.at[base + 1], bufs[1], isems[1])

    slot(0, 0, osem_wait=False)
    slot(0, 1, osem_wait=False)
    slot(0, 2)
    slot(0, 3)

    def group_body(g, carry):
        for k in range(NBUF):
            slot(g, k)
        return carry
    lax.fori_loop(1, GROUPS - 1, group_body, 0)

    slot(GROUPS - 1, 0)
    slot(GROUPS - 1, 1)
    slot(GROUPS - 1, 2, prefetch=False)
    slot(GROUPS - 1, 3, prefetch=False)

    for k in range(NBUF):
        t = base + (GROUPS - 1) * NBUF + k
        pltpu.make_async_copy(bufs[k], out_hbm.at[t], osems[k]).wait()


def kernel(x, pos_table):
    xf = x.reshape(B, ROW)
    pf = pos_table.reshape(ROW)
    out = _sc_add(xf, pf)
    return out.reshape(B, SEQ, DIM)
